# Initial kernel scaffold; baseline (speedup 1.0000x reference)
import jax
import jax.numpy as jnp
from jax.experimental import pallas as pl


def kernel(semantic_global, semantic_local, visual_feat, params):
    raise NotImplementedError("write your pallas kernel here")


# speedup vs baseline: 21.0502x; 21.0502x over previous
import functools

import jax
import jax.numpy as jnp
import numpy as np
from jax.experimental import pallas as pl
from jax.experimental.pallas import tpu as pltpu

D = 128
H = 8
B = 16384
RES = 2048
K = 80

ROWS = 256

_WNAMES = ['gu1', 'gu2', 'gd1', 'gd2', 'lu1', 'lu2', 'ld1', 'ld2', 'v', 'o']
(I_GU1, I_GU2, I_GD1, I_GD2, I_LU1, I_LU2, I_LD1, I_LD2, I_V, I_O) = range(10)


def _l2norm_rows(x):
    n = jnp.sqrt(jnp.sum(x * x, axis=1, keepdims=True))
    return x / jnp.maximum(n, 1e-12)


def _tc_body(sg_ref, sl_ref, vf_ref, w_ref, b_ref, tri_ref,
             fused_ref, fff_ref, f2_ref):
    def mm(x, i):
        return jnp.dot(x, w_ref[i], preferred_element_type=jnp.float32) + b_ref[i]

    sg = sg_ref[...]
    sl = sl_ref[...]

    u = mm(jax.nn.relu(mm(sg, I_GU1)), I_GU2)
    d = mm(jax.nn.relu(mm(sg, I_GD1)), I_GD2)
    sg2 = jax.nn.sigmoid(u) * d

    u = mm(jax.nn.relu(mm(sl, I_LU1)), I_LU2)
    d = mm(jax.nn.relu(mm(sl, I_LD1)), I_LD2)
    sl2 = jax.nn.sigmoid(u) * d

    go = mm(mm(sg2, I_V), I_O)
    lo = mm(mm(sl2, I_V), I_O)

    f1 = _l2norm_rows(go)
    f2 = _l2norm_rows(lo)
    fff = jax.nn.sigmoid(f1) * f2

    a_int = jax.lax.bitcast_convert_type(jnp.abs(fff), jnp.int32)
    t = jnp.zeros((fff.shape[0], 1), jnp.int32)
    for bit in range(30, -1, -1):
        cand = t | (1 << bit)
        cnt = jnp.sum((a_int >= cand).astype(jnp.int32), axis=1, keepdims=True)
        t = jnp.where(cnt >= K, cand, t)

    gt = a_int > t
    eq = a_int == t
    n_gt = jnp.sum(gt.astype(jnp.float32), axis=1, keepdims=True)
    prefix = jnp.dot(eq.astype(jnp.float32), tri_ref[...],
                     preferred_element_type=jnp.float32)
    keep = gt | (eq & (prefix <= (K - n_gt)))
    sparse = jnp.where(keep, fff, 0.0)

    fused_ref[:, :RES] = vf_ref[...]
    fused_ref[:, RES:] = sparse
    fff_ref[...] = fff
    f2_ref[...] = f2


def _run_tc(sg, sl, vf, wstack, bstack, tri, *, interpret=False):
    grid = (B // ROWS,)
    return pl.pallas_call(
        _tc_body,
        grid=grid,
        in_specs=[
            pl.BlockSpec((ROWS, D), lambda i: (i, 0)),
            pl.BlockSpec((ROWS, D), lambda i: (i, 0)),
            pl.BlockSpec((ROWS, RES), lambda i: (i, 0)),
            pl.BlockSpec((10, D, D), lambda i: (0, 0, 0)),
            pl.BlockSpec((10, 1, D), lambda i: (0, 0, 0)),
            pl.BlockSpec((D, D), lambda i: (0, 0)),
        ],
        out_specs=[
            pl.BlockSpec((ROWS, RES + D), lambda i: (i, 0)),
            pl.BlockSpec((ROWS, D), lambda i: (i, 0)),
            pl.BlockSpec((ROWS, D), lambda i: (i, 0)),
        ],
        out_shape=[
            jax.ShapeDtypeStruct((B, RES + D), jnp.float32),
            jax.ShapeDtypeStruct((B, D), jnp.float32),
            jax.ShapeDtypeStruct((B, D), jnp.float32),
        ],
        compiler_params=pltpu.CompilerParams(
            dimension_semantics=("arbitrary",),
        ),
        interpret=interpret,
    )(sg, sl, vf, wstack, bstack, tri)


def kernel(semantic_global, semantic_local, visual_feat, params):
    wstack = jnp.stack([params['W_' + n].T for n in _WNAMES])
    bstack = jnp.stack([params['b_' + n] for n in _WNAMES])[:, None, :]
    tri = jnp.triu(jnp.ones((D, D), jnp.float32))
    fused, fff, f2 = _run_tc(semantic_global, semantic_local, visual_feat,
                             wstack, bstack, tri)
    return fused, fff, f2


# speedup vs baseline: 21.3115x; 1.0124x over previous
import functools

import jax
import jax.numpy as jnp
import numpy as np
from jax.experimental import pallas as pl
from jax.experimental.pallas import tpu as pltpu

D = 128
H = 8
B = 16384
RES = 2048
K = 80

ROWS = 512


def _l2norm_rows(x):
    n = jnp.sqrt(jnp.sum(x * x, axis=1, keepdims=True))
    return x / jnp.maximum(n, 1e-12)


def _tc_body(sg_ref, sl_ref, vf_ref, w1g_ref, w1l_ref, w2g_ref, w2l_ref,
             wv_ref, wo_ref, b256_ref, b128_ref, tri_ref,
             fused_ref, fff_ref, f2_ref):
    f32 = jnp.float32

    def dot(x, w):
        return jnp.dot(x, w, preferred_element_type=f32)

    sg = sg_ref[...]
    sl = sl_ref[...]

    h = jax.nn.relu(dot(sg, w1g_ref[...]) + b256_ref[0])
    h = dot(h, w2g_ref[...]) + b256_ref[1]
    sg2 = jax.nn.sigmoid(h[:, :D]) * h[:, D:]

    h = jax.nn.relu(dot(sl, w1l_ref[...]) + b256_ref[2])
    h = dot(h, w2l_ref[...]) + b256_ref[3]
    sl2 = jax.nn.sigmoid(h[:, :D]) * h[:, D:]

    z = jnp.concatenate([sg2, sl2], axis=0)
    z = dot(dot(z, wv_ref[...]) + b128_ref[0], wo_ref[...]) + b128_ref[1]
    f1 = _l2norm_rows(z[:ROWS])
    f2 = _l2norm_rows(z[ROWS:])
    fff = jax.nn.sigmoid(f1) * f2

    tri = tri_ref[...]
    ones = tri[:, D - 1:D]
    a_int = jax.lax.bitcast_convert_type(jnp.abs(fff), jnp.int32)
    t = jnp.zeros((ROWS, 1), jnp.int32)
    for bit in range(30, -1, -1):
        cand = t | (1 << bit)
        ge = jnp.where(a_int >= cand, 1.0, 0.0)
        cnt = dot(ge, ones)
        t = jnp.where(cnt >= K, cand, t)

    gt = a_int > t
    eq = a_int == t
    eq_f = jnp.where(eq, 1.0, 0.0)
    n_gt = dot(jnp.where(gt, 1.0, 0.0), ones)
    prefix = dot(eq_f, tri)
    keep = gt | (eq & (prefix <= (K - n_gt)))
    sparse = jnp.where(keep, fff, 0.0)

    fused_ref[:, :RES] = vf_ref[...]
    fused_ref[:, RES:] = sparse
    fff_ref[...] = fff
    f2_ref[...] = f2


def _run_tc(sg, sl, vf, w1g, w1l, w2g, w2l, wv, wo, b256, b128, tri,
            *, interpret=False):
    grid = (B // ROWS,)
    row_spec = lambda c: pl.BlockSpec((ROWS, c), lambda i: (i, 0))
    full2 = lambda a, b: pl.BlockSpec((a, b), lambda i: (0, 0))
    full3 = lambda a, b, c: pl.BlockSpec((a, b, c), lambda i: (0, 0, 0))
    return pl.pallas_call(
        _tc_body,
        grid=grid,
        in_specs=[
            row_spec(D), row_spec(D), row_spec(RES),
            full2(D, 2 * D), full2(D, 2 * D),
            full2(2 * D, 2 * D), full2(2 * D, 2 * D),
            full2(D, D), full2(D, D),
            full3(4, 1, 2 * D), full3(2, 1, D),
            full2(D, D),
        ],
        out_specs=[
            row_spec(RES + D), row_spec(D), row_spec(D),
        ],
        out_shape=[
            jax.ShapeDtypeStruct((B, RES + D), jnp.float32),
            jax.ShapeDtypeStruct((B, D), jnp.float32),
            jax.ShapeDtypeStruct((B, D), jnp.float32),
        ],
        compiler_params=pltpu.CompilerParams(
            dimension_semantics=("arbitrary",),
        ),
        interpret=interpret,
    )(sg, sl, vf, w1g, w1l, w2g, w2l, wv, wo, b256, b128, tri)


def kernel(semantic_global, semantic_local, visual_feat, params):
    p = params
    f32 = jnp.float32

    def blockdiag(a, b):
        z = jnp.zeros((D, D), f32)
        return jnp.block([[a, z], [z, b]])

    w1g = jnp.concatenate([p['W_gu1'].T, p['W_gd1'].T], axis=1)
    w1l = jnp.concatenate([p['W_lu1'].T, p['W_ld1'].T], axis=1)
    w2g = blockdiag(p['W_gu2'].T, p['W_gd2'].T)
    w2l = blockdiag(p['W_lu2'].T, p['W_ld2'].T)
    wv = p['W_v'].T
    wo = p['W_o'].T
    b256 = jnp.stack([
        jnp.concatenate([p['b_gu1'], p['b_gd1']]),
        jnp.concatenate([p['b_gu2'], p['b_gd2']]),
        jnp.concatenate([p['b_lu1'], p['b_ld1']]),
        jnp.concatenate([p['b_lu2'], p['b_ld2']]),
    ])[:, None, :]
    b128 = jnp.stack([p['b_v'], p['b_o']])[:, None, :]
    tri = jnp.triu(jnp.ones((D, D), f32))
    fused, fff, f2 = _run_tc(semantic_global, semantic_local, visual_feat,
                             w1g, w1l, w2g, w2l, wv, wo, b256, b128, tri)
    return fused, fff, f2


# speedup vs baseline: 41.0481x; 1.9261x over previous
import functools

import jax
import jax.numpy as jnp
import numpy as np
from jax.experimental import pallas as pl
from jax.experimental.pallas import tpu as pltpu

D = 128
H = 8
B = 16384
RES = 2048
K = 80

ROWS = 512


def _l2norm_cols(x):
    n = jnp.sqrt(jnp.sum(x * x, axis=0, keepdims=True))
    return x / jnp.maximum(n, 1e-12)


def _tc_body(sg_ref, sl_ref, vf_ref, w1g_ref, w1l_ref, w2g_ref, w2l_ref,
             wv_ref, wo_ref, b1g_ref, b1l_ref, b2g_ref, b2l_ref,
             bv_ref, bo_ref, tril_ref,
             fused_ref, fff_ref, f2_ref):
    f32 = jnp.float32

    def dot(w, x):
        return jnp.dot(w, x, preferred_element_type=f32)

    sgT = sg_ref[...].T
    slT = sl_ref[...].T

    h = jax.nn.relu(dot(w1g_ref[...], sgT) + b1g_ref[...])
    h = dot(w2g_ref[...], h) + b2g_ref[...]
    sg2T = jax.nn.sigmoid(h[:D]) * h[D:]

    h = jax.nn.relu(dot(w1l_ref[...], slT) + b1l_ref[...])
    h = dot(w2l_ref[...], h) + b2l_ref[...]
    sl2T = jax.nn.sigmoid(h[:D]) * h[D:]

    zT = jnp.concatenate([sg2T, sl2T], axis=1)
    zT = dot(wo_ref[...], dot(wv_ref[...], zT) + bv_ref[...]) + bo_ref[...]
    f1T = _l2norm_cols(zT[:, :ROWS])
    f2T = _l2norm_cols(zT[:, ROWS:])
    fffT = jax.nn.sigmoid(f1T) * f2T

    aT = jax.lax.bitcast_convert_type(jnp.abs(fffT), jnp.int32)
    t = jnp.zeros((1, ROWS), jnp.int32)
    for bit in range(29, -1, -1):
        cand = t | (1 << bit)
        ge = jnp.where(aT >= cand, 1.0, 0.0)
        cnt = jnp.sum(ge, axis=0, keepdims=True)
        t = jnp.where(cnt >= K, cand, t)

    gt = aT > t
    eq = aT == t
    n_gt = jnp.sum(jnp.where(gt, 1.0, 0.0), axis=0, keepdims=True)
    prefix = dot(tril_ref[...], jnp.where(eq, 1.0, 0.0))
    keep = gt | (eq & (prefix <= (K - n_gt)))
    sparseT = jnp.where(keep, fffT, 0.0)

    fused_ref[:, :RES] = vf_ref[...]
    fused_ref[:, RES:] = sparseT.T
    fff_ref[...] = fffT.T
    f2_ref[...] = f2T.T


def _run_tc(sg, sl, vf, w1g, w1l, w2g, w2l, wv, wo,
            b1g, b1l, b2g, b2l, bv, bo, tril, *, interpret=False):
    grid = (B // ROWS,)
    row_spec = lambda c: pl.BlockSpec((ROWS, c), lambda i: (i, 0))
    full2 = lambda a, b: pl.BlockSpec((a, b), lambda i: (0, 0))
    return pl.pallas_call(
        _tc_body,
        grid=grid,
        in_specs=[
            row_spec(D), row_spec(D), row_spec(RES),
            full2(2 * D, D), full2(2 * D, D),
            full2(2 * D, 2 * D), full2(2 * D, 2 * D),
            full2(D, D), full2(D, D),
            full2(2 * D, 1), full2(2 * D, 1),
            full2(2 * D, 1), full2(2 * D, 1),
            full2(D, 1), full2(D, 1),
            full2(D, D),
        ],
        out_specs=[
            row_spec(RES + D), row_spec(D), row_spec(D),
        ],
        out_shape=[
            jax.ShapeDtypeStruct((B, RES + D), jnp.float32),
            jax.ShapeDtypeStruct((B, D), jnp.float32),
            jax.ShapeDtypeStruct((B, D), jnp.float32),
        ],
        compiler_params=pltpu.CompilerParams(
            dimension_semantics=("arbitrary",),
        ),
        interpret=interpret,
    )(sg, sl, vf, w1g, w1l, w2g, w2l, wv, wo,
      b1g, b1l, b2g, b2l, bv, bo, tril)


def kernel(semantic_global, semantic_local, visual_feat, params):
    p = params
    f32 = jnp.float32

    def blockdiag(a, b):
        z = jnp.zeros((D, D), f32)
        return jnp.block([[a, z], [z, b]])

    w1g = jnp.concatenate([p['W_gu1'], p['W_gd1']], axis=0)
    w1l = jnp.concatenate([p['W_lu1'], p['W_ld1']], axis=0)
    w2g = blockdiag(p['W_gu2'], p['W_gd2'])
    w2l = blockdiag(p['W_lu2'], p['W_ld2'])
    wv = p['W_v']
    wo = p['W_o']
    b1g = jnp.concatenate([p['b_gu1'], p['b_gd1']])[:, None]
    b1l = jnp.concatenate([p['b_lu1'], p['b_ld1']])[:, None]
    b2g = jnp.concatenate([p['b_gu2'], p['b_gd2']])[:, None]
    b2l = jnp.concatenate([p['b_lu2'], p['b_ld2']])[:, None]
    bv = p['b_v'][:, None]
    bo = p['b_o'][:, None]
    tril = jnp.tril(jnp.ones((D, D), f32))
    fused, fff, f2 = _run_tc(semantic_global, semantic_local, visual_feat,
                             w1g, w1l, w2g, w2l, wv, wo,
                             b1g, b1l, b2g, b2l, bv, bo, tril)
    return fused, fff, f2


# speedup vs baseline: 46.1947x; 1.1254x over previous
import functools

import jax
import jax.numpy as jnp
import numpy as np
from jax.experimental import pallas as pl
from jax.experimental.pallas import tpu as pltpu

D = 128
H = 8
B = 16384
RES = 2048
K = 80

ROWS = 1024


def _l2norm_cols(x):
    n = jnp.sqrt(jnp.sum(x * x, axis=0, keepdims=True))
    return x / jnp.maximum(n, 1e-12)


def _tc_body(sg_ref, sl_ref, vf_ref, w1g_ref, w1l_ref, w2g_ref, w2l_ref,
             wv_ref, wo_ref, b1g_ref, b1l_ref, b2g_ref, b2l_ref,
             bv_ref, bo_ref, tril_ref,
             fused_ref, fff_ref, f2_ref):
    f32 = jnp.float32

    def dot(w, x):
        return jnp.dot(w, x, preferred_element_type=f32)

    def dot_rt(w, x):
        return jax.lax.dot_general(w, x, (((1,), (1,)), ((), ())),
                                   preferred_element_type=f32)

    h = jax.nn.relu(dot_rt(w1g_ref[...], sg_ref[...]) + b1g_ref[...])
    h = dot(w2g_ref[...], h) + b2g_ref[...]
    sg2T = jax.nn.sigmoid(h[:D]) * h[D:]

    h = jax.nn.relu(dot_rt(w1l_ref[...], sl_ref[...]) + b1l_ref[...])
    h = dot(w2l_ref[...], h) + b2l_ref[...]
    sl2T = jax.nn.sigmoid(h[:D]) * h[D:]

    zT = jnp.concatenate([sg2T, sl2T], axis=1)
    zT = dot(wo_ref[...], dot(wv_ref[...], zT) + bv_ref[...]) + bo_ref[...]
    f1T = _l2norm_cols(zT[:, :ROWS])
    f2T = _l2norm_cols(zT[:, ROWS:])
    fffT = jax.nn.sigmoid(f1T) * f2T

    aT = jax.lax.bitcast_convert_type(jnp.abs(fffT), jnp.int32)
    t = jnp.zeros((1, ROWS), jnp.int32)
    for bit in range(29, -1, -1):
        cand = t | (1 << bit)
        ge = jnp.where(aT >= cand, 1.0, 0.0)
        cnt = jnp.sum(ge, axis=0, keepdims=True)
        t = jnp.where(cnt >= K, cand, t)

    gt = aT > t
    eq = aT == t
    n_gt = jnp.sum(jnp.where(gt, 1.0, 0.0), axis=0, keepdims=True)
    prefix = dot(tril_ref[...], jnp.where(eq, 1.0, 0.0))
    keep = gt | (eq & (prefix <= (K - n_gt)))
    sparseT = jnp.where(keep, fffT, 0.0)

    fused_ref[:, :RES] = vf_ref[...]
    fused_ref[:, RES:] = sparseT.T
    fff_ref[...] = fffT.T
    f2_ref[...] = f2T.T


def _run_tc(sg, sl, vf, w1g, w1l, w2g, w2l, wv, wo,
            b1g, b1l, b2g, b2l, bv, bo, tril, *, interpret=False):
    grid = (B // ROWS,)
    row_spec = lambda c: pl.BlockSpec((ROWS, c), lambda i: (i, 0))
    full2 = lambda a, b: pl.BlockSpec((a, b), lambda i: (0, 0))
    return pl.pallas_call(
        _tc_body,
        grid=grid,
        in_specs=[
            row_spec(D), row_spec(D), row_spec(RES),
            full2(2 * D, D), full2(2 * D, D),
            full2(2 * D, 2 * D), full2(2 * D, 2 * D),
            full2(D, D), full2(D, D),
            full2(2 * D, 1), full2(2 * D, 1),
            full2(2 * D, 1), full2(2 * D, 1),
            full2(D, 1), full2(D, 1),
            full2(D, D),
        ],
        out_specs=[
            row_spec(RES + D), row_spec(D), row_spec(D),
        ],
        out_shape=[
            jax.ShapeDtypeStruct((B, RES + D), jnp.float32),
            jax.ShapeDtypeStruct((B, D), jnp.float32),
            jax.ShapeDtypeStruct((B, D), jnp.float32),
        ],
        compiler_params=pltpu.CompilerParams(
            dimension_semantics=("arbitrary",),
        ),
        interpret=interpret,
    )(sg, sl, vf, w1g, w1l, w2g, w2l, wv, wo,
      b1g, b1l, b2g, b2l, bv, bo, tril)


def kernel(semantic_global, semantic_local, visual_feat, params):
    p = params
    f32 = jnp.float32

    def blockdiag(a, b):
        z = jnp.zeros((D, D), f32)
        return jnp.block([[a, z], [z, b]])

    w1g = jnp.concatenate([p['W_gu1'], p['W_gd1']], axis=0)
    w1l = jnp.concatenate([p['W_lu1'], p['W_ld1']], axis=0)
    w2g = blockdiag(p['W_gu2'], p['W_gd2'])
    w2l = blockdiag(p['W_lu2'], p['W_ld2'])
    wv = p['W_v']
    wo = p['W_o']
    b1g = jnp.concatenate([p['b_gu1'], p['b_gd1']])[:, None]
    b1l = jnp.concatenate([p['b_lu1'], p['b_ld1']])[:, None]
    b2g = jnp.concatenate([p['b_gu2'], p['b_gd2']])[:, None]
    b2l = jnp.concatenate([p['b_lu2'], p['b_ld2']])[:, None]
    bv = p['b_v'][:, None]
    bo = p['b_o'][:, None]
    tril = jnp.tril(jnp.ones((D, D), f32))
    fused, fff, f2 = _run_tc(semantic_global, semantic_local, visual_feat,
                             w1g, w1l, w2g, w2l, wv, wo,
                             b1g, b1l, b2g, b2l, bv, bo, tril)
    return fused, fff, f2
